# baseline (speedup 1.0000x reference)
import functools

import jax
import jax.numpy as jnp
from jax import lax
from jax.experimental import pallas as pl
from jax.experimental.pallas import tpu as pltpu
from jax.experimental.pallas import tpu_sc as plsc

NUM_CLASSES = 100000
HIDDEN_SIZE = 64
DROPOUT_PROB = 0.1
BATCH = 16384

NC, NS = 2, 16
NW = NC * NS
B_PER_W = BATCH // NW

_mesh = plsc.VectorSubcoreMesh(
    core_axis_name="c", subcore_axis_name="s", num_cores=NC, num_subcores=NS
)


@functools.partial(
    pl.kernel,
    out_type=jax.ShapeDtypeStruct((BATCH, HIDDEN_SIZE), jnp.float32),
    mesh=_mesh,
    compiler_params=pltpu.CompilerParams(
        skip_device_barrier=True,
        disable_bounds_checks=True,
        disable_semaphore_checks=True,
    ),
    scratch_types=[
        pltpu.VMEM((B_PER_W,), jnp.int32),
        pltpu.VMEM((B_PER_W, HIDDEN_SIZE), jnp.float32),
        pltpu.SemaphoreType.DMA,
    ],
)
def _gather_rows(labels_hbm, table_hbm, out_hbm, idx_v, rows_v, sem):
    wid = lax.axis_index("s") * NC + lax.axis_index("c")
    base = wid * B_PER_W
    pltpu.sync_copy(labels_hbm.at[pl.ds(base, B_PER_W)], idx_v)

    def body(k, _):
        chunk = idx_v[pl.ds(k * 16, 16)]
        for j in range(16):
            pltpu.async_copy(
                table_hbm.at[pl.ds(chunk[j], 1)],
                rows_v.at[pl.ds(k * 16 + j, 1)],
                sem,
            )
        return ()

    lax.fori_loop(0, B_PER_W // 16, body, ())
    pltpu.make_async_copy(
        table_hbm.at[pl.ds(0, B_PER_W)], rows_v, sem
    ).wait()
    pltpu.sync_copy(rows_v, out_hbm.at[pl.ds(base, B_PER_W)])


def kernel(labels, train, embedding_table):
    drop_key = jax.random.key(1)
    drop_ids = jax.random.uniform(drop_key, (labels.shape[0],)) < DROPOUT_PROB
    active = (jnp.asarray(train) != 0) & drop_ids
    labels = jnp.where(active, NUM_CLASSES, labels).astype(jnp.int32)
    return _gather_rows(labels, embedding_table)


# speedup vs baseline: 1.0010x; 1.0010x over previous
import functools

import jax
import jax.numpy as jnp
from jax import lax
from jax.experimental import pallas as pl
from jax.experimental.pallas import tpu as pltpu
from jax.experimental.pallas import tpu_sc as plsc

NUM_CLASSES = 100000
HIDDEN_SIZE = 64
DROPOUT_PROB = 0.1
BATCH = 16384

NC, NS = 2, 16
NW = NC * NS
B_PER_W = BATCH // NW

_mesh = plsc.VectorSubcoreMesh(
    core_axis_name="c", subcore_axis_name="s", num_cores=NC, num_subcores=NS
)


@functools.partial(
    pl.kernel,
    out_type=jax.ShapeDtypeStruct((BATCH, HIDDEN_SIZE), jnp.float32),
    mesh=_mesh,
    compiler_params=pltpu.CompilerParams(
        use_tc_tiling_on_sc=True,
        skip_device_barrier=True,
        disable_bounds_checks=True,
        disable_semaphore_checks=True,
    ),
    scratch_types=[
        pltpu.VMEM((B_PER_W,), jnp.int32),
        pltpu.VMEM((B_PER_W, HIDDEN_SIZE), jnp.float32),
        pltpu.SemaphoreType.DMA,
    ],
)
def _gather_rows(labels_hbm, table_hbm, out_hbm, idx_v, rows_v, sem):
    wid = lax.axis_index("s") * NC + lax.axis_index("c")
    base = wid * B_PER_W
    pltpu.sync_copy(labels_hbm.at[pl.ds(base, B_PER_W)], idx_v)

    def body(k, _):
        chunk = idx_v[pl.ds(k * 16, 16)]
        for j in range(16):
            pltpu.async_copy(
                table_hbm.at[pl.ds(chunk[j], 1)],
                rows_v.at[pl.ds(k * 16 + j, 1)],
                sem,
            )
        return ()

    lax.fori_loop(0, B_PER_W // 16, body, ())
    pltpu.make_async_copy(
        table_hbm.at[pl.ds(0, B_PER_W)], rows_v, sem
    ).wait()
    pltpu.sync_copy(rows_v, out_hbm.at[pl.ds(base, B_PER_W)])


def kernel(labels, train, embedding_table):
    drop_key = jax.random.key(1)
    drop_ids = jax.random.uniform(drop_key, (labels.shape[0],)) < DROPOUT_PROB
    active = (jnp.asarray(train) != 0) & drop_ids
    labels = jnp.where(active, NUM_CLASSES, labels).astype(jnp.int32)
    return _gather_rows(labels, embedding_table)


# speedup vs baseline: 1.2326x; 1.2314x over previous
import functools

import jax
import jax.numpy as jnp
from jax import lax
from jax.experimental import pallas as pl
from jax.experimental.pallas import tpu as pltpu
from jax.experimental.pallas import tpu_sc as plsc

NUM_CLASSES = 100000
HIDDEN_SIZE = 64
DROPOUT_PROB = 0.1
BATCH = 16384

NC, NS = 2, 16
NW = NC * NS
ROWS_PER_W = HIDDEN_SIZE // NW
BHALF = BATCH // 2

_mesh = plsc.VectorSubcoreMesh(
    core_axis_name="c", subcore_axis_name="s", num_cores=NC, num_subcores=NS
)


@functools.partial(
    pl.kernel,
    out_type=jax.ShapeDtypeStruct((HIDDEN_SIZE, BATCH), jnp.float32),
    mesh=_mesh,
    compiler_params=pltpu.CompilerParams(
        use_tc_tiling_on_sc=True,
        needs_layout_passes=False,
        skip_device_barrier=True,
        disable_bounds_checks=True,
        disable_semaphore_checks=True,
    ),
    scratch_types=[
        pltpu.VMEM((NUM_CLASSES + 1,), jnp.float32),
        pltpu.VMEM((BHALF,), jnp.int32),
        pltpu.VMEM((BHALF,), jnp.float32),
    ],
)
def _gather_cols(labels_hbm, table_t_hbm, out_t_hbm, row_v, idx_v, out_v):
    wid = lax.axis_index("s") * NC + lax.axis_index("c")
    for r in range(ROWS_PER_W):
        c = wid * ROWS_PER_W + r
        pltpu.sync_copy(table_t_hbm.at[c], row_v)
        for h in range(2):
            pltpu.sync_copy(labels_hbm.at[pl.ds(h * BHALF, BHALF)], idx_v)

            def body(k, _):
                idx16 = idx_v[pl.ds(k * 16, 16)]
                out_v[pl.ds(k * 16, 16)] = plsc.load_gather(row_v, [idx16])
                return ()

            lax.fori_loop(0, BHALF // 16, body, (), unroll=8)
            pltpu.sync_copy(out_v, out_t_hbm.at[c, pl.ds(h * BHALF, BHALF)])


def kernel(labels, train, embedding_table):
    drop_key = jax.random.key(1)
    drop_ids = jax.random.uniform(drop_key, (labels.shape[0],)) < DROPOUT_PROB
    active = (jnp.asarray(train) != 0) & drop_ids
    labels = jnp.where(active, NUM_CLASSES, labels).astype(jnp.int32)
    out_t = _gather_cols(labels, embedding_table.T)
    return out_t.T


# speedup vs baseline: 1.6689x; 1.3539x over previous
import functools

import jax
import jax.numpy as jnp
from jax import lax
from jax.experimental import pallas as pl
from jax.experimental.pallas import tpu as pltpu
from jax.experimental.pallas import tpu_sc as plsc

NUM_CLASSES = 100000
HIDDEN_SIZE = 64
DROPOUT_PROB = 0.1
BATCH = 16384

NC, NS = 2, 16
NW = NC * NS
ROWS_PER_W = HIDDEN_SIZE // NW
BHALF = BATCH // 2

_mesh = plsc.VectorSubcoreMesh(
    core_axis_name="c", subcore_axis_name="s", num_cores=NC, num_subcores=NS
)


@functools.partial(
    pl.kernel,
    out_type=jax.ShapeDtypeStruct((HIDDEN_SIZE, BATCH), jnp.float32),
    mesh=_mesh,
    compiler_params=pltpu.CompilerParams(
        use_tc_tiling_on_sc=True,
        needs_layout_passes=False,
        skip_device_barrier=True,
        disable_bounds_checks=True,
        disable_semaphore_checks=True,
    ),
    scratch_types=[
        pltpu.VMEM((NUM_CLASSES + 1,), jnp.float32),
        pltpu.VMEM((BHALF,), jnp.int32),
        pltpu.VMEM((BHALF,), jnp.float32),
    ],
)
def _gather_cols(labels_hbm, table_t_hbm, out_t_hbm, row_v, idx_v, out_v):
    wid = lax.axis_index("s") * NC + lax.axis_index("c")
    for r in range(ROWS_PER_W):
        c = wid * ROWS_PER_W + r
        pltpu.sync_copy(table_t_hbm.at[c], row_v)
        for h in range(2):
            pltpu.sync_copy(labels_hbm.at[pl.ds(h * BHALF, BHALF)], idx_v)

            @plsc.parallel_loop(0, BHALF, step=16, unroll=8)
            def _(k):
                idx16 = idx_v[pl.ds(k, 16)]
                out_v[pl.ds(k, 16)] = plsc.load_gather(row_v, [idx16])
            pltpu.sync_copy(out_v, out_t_hbm.at[c, pl.ds(h * BHALF, BHALF)])


def kernel(labels, train, embedding_table):
    drop_key = jax.random.key(1)
    drop_ids = jax.random.uniform(drop_key, (labels.shape[0],)) < DROPOUT_PROB
    active = (jnp.asarray(train) != 0) & drop_ids
    labels = jnp.where(active, NUM_CLASSES, labels).astype(jnp.int32)
    out_t = _gather_cols(labels, embedding_table.T)
    return out_t.T
